# Initial kernel scaffold; baseline (speedup 1.0000x reference)
import jax
import jax.numpy as jnp
from jax.experimental import pallas as pl


def kernel(Q, K, V):
    raise NotImplementedError("write your pallas kernel here")


# speedup vs baseline: 3.8322x; 3.8322x over previous
import functools
import math

import jax
import jax.numpy as jnp
from jax.experimental import pallas as pl


def _fwd_kernel(cos_ref, sin_ref, xq_ref, xk_ref,
                cq_ref, sq_ref, ck_ref, sk_ref):
    k = pl.program_id(2)

    @pl.when(k == 0)
    def _():
        cq_ref[...] = jnp.zeros_like(cq_ref)
        sq_ref[...] = jnp.zeros_like(sq_ref)
        ck_ref[...] = jnp.zeros_like(ck_ref)
        sk_ref[...] = jnp.zeros_like(sk_ref)

    c = cos_ref[...]
    s = sin_ref[...]
    xq = xq_ref[...]
    xk = xk_ref[...]
    cq_ref[...] += jnp.dot(c, xq, preferred_element_type=jnp.float32, precision=jax.lax.Precision.HIGHEST)
    sq_ref[...] += jnp.dot(s, xq, preferred_element_type=jnp.float32, precision=jax.lax.Precision.HIGHEST)
    ck_ref[...] += jnp.dot(c, xk, preferred_element_type=jnp.float32, precision=jax.lax.Precision.HIGHEST)
    sk_ref[...] += jnp.dot(s, xk, preferred_element_type=jnp.float32, precision=jax.lax.Precision.HIGHEST)


def _inv_kernel(inv_l, cos_ref, sin_ref, cq_ref, sq_ref, ck_ref, sk_ref,
                corr_ref):
    k = pl.program_id(2)

    @pl.when(k == 0)
    def _():
        corr_ref[...] = jnp.zeros_like(corr_ref)

    cq = cq_ref[...]
    sq = sq_ref[...]
    ck = ck_ref[...]
    sk = sk_ref[...]
    pr = (cq * ck + sq * sk) * inv_l
    pi = (cq * sk - sq * ck) * inv_l
    corr_ref[...] += (
        jnp.dot(cos_ref[...], pr, preferred_element_type=jnp.float32, precision=jax.lax.Precision.HIGHEST)
        - jnp.dot(sin_ref[...], pi, preferred_element_type=jnp.float32, precision=jax.lax.Precision.HIGHEST))


def _topk_kernel(seq_len, topk, corr_ref, w_ref, i_ref):
    c = corr_ref[...]
    tn = c.shape[1]
    iota = jax.lax.broadcasted_iota(jnp.int32, (seq_len, tn), 0)

    vals = []
    idxs = []
    for _ in range(topk):
        m = jnp.max(c, axis=0, keepdims=True)
        a = jnp.min(jnp.where(c == m, iota, seq_len), axis=0,
                    keepdims=True)
        vals.append(m)
        idxs.append(a)
        c = jnp.where(iota == a, -jnp.inf, c)

    w = jnp.concatenate(vals, axis=0)
    wmax = jnp.max(w, axis=0, keepdims=True)
    e = jnp.exp(w - wmax)
    w_ref[...] = e / jnp.sum(e, axis=0, keepdims=True)
    i_ref[...] = jnp.concatenate(idxs, axis=0)


def _gather_kernel(seq_len, v_ref, w_ref, i_ref, o_ref):
    i = pl.program_id(1)

    @pl.when(i == 0)
    def _():
        o_ref[...] = jnp.zeros_like(o_ref)

    u = v_ref[...]
    tn = u.shape[1]
    idx = i_ref[0]
    nbits = seq_len.bit_length() - 1
    for bit in range(nbits):
        s = 1 << bit
        tail = jnp.broadcast_to(u[seq_len - 1:, :], (s, tn))
        shifted = jnp.concatenate([u[s:, :], tail], axis=0)
        take = ((idx >> bit) & 1) == 1
        u = jnp.where(take, shifted, u)
    o_ref[...] += w_ref[0] * u


def kernel(Q, K, V):
    B, L, D = Q.shape
    cols = B * D
    topk = int(2 * math.log(L))
    f32 = jnp.float32

    xq = jnp.transpose(Q, (1, 0, 2)).reshape(L, cols)
    xk = jnp.transpose(K, (1, 0, 2)).reshape(L, cols)
    xv = jnp.transpose(V, (1, 0, 2)).reshape(L, cols)

    t = jnp.arange(L, dtype=jnp.int32)
    phase = (t[:, None] * t[None, :]) % L
    ang = phase.astype(f32) * f32(2.0 * math.pi / L)
    cos_m = jnp.cos(ang)
    sin_m = jnp.sin(ang)

    tm = min(512, L)
    tk = min(512, L)
    tn = min(512, cols)
    grid = (L // tm, cols // tn, L // tk)
    mat_specs = dict(
        cos=pl.BlockSpec((tm, tk), lambda i, j, k: (i, k)),
        x=pl.BlockSpec((tk, tn), lambda i, j, k: (k, j)),
        out=pl.BlockSpec((tm, tn), lambda i, j, k: (i, j)),
    )
    lcol = jax.ShapeDtypeStruct((L, cols), f32)

    cq, sq, ck, sk = pl.pallas_call(
        _fwd_kernel,
        grid=grid,
        in_specs=[mat_specs["cos"], mat_specs["cos"],
                  mat_specs["x"], mat_specs["x"]],
        out_specs=[mat_specs["out"]] * 4,
        out_shape=[lcol] * 4,
    )(cos_m, sin_m, xq, xk)

    corr = pl.pallas_call(
        functools.partial(_inv_kernel, float(1.0 / L)),
        grid=grid,
        in_specs=[mat_specs["cos"], mat_specs["cos"]] + [mat_specs["x"]] * 4,
        out_specs=mat_specs["out"],
        out_shape=lcol,
    )(cos_m, sin_m, cq, sq, ck, sk)

    tn2 = min(256, cols)
    wk, ik = pl.pallas_call(
        functools.partial(_topk_kernel, L, topk),
        grid=(cols // tn2,),
        in_specs=[pl.BlockSpec((L, tn2), lambda j: (0, j))],
        out_specs=[pl.BlockSpec((topk, tn2), lambda j: (0, j))] * 2,
        out_shape=[jax.ShapeDtypeStruct((topk, cols), f32),
                   jax.ShapeDtypeStruct((topk, cols), jnp.int32)],
    )(corr)

    wk3 = wk.reshape(topk, 1, cols)
    ik3 = ik.reshape(topk, 1, cols)
    out_cols = pl.pallas_call(
        functools.partial(_gather_kernel, L),
        grid=(cols // tn2, topk),
        in_specs=[pl.BlockSpec((L, tn2), lambda j, i: (0, j)),
                  pl.BlockSpec((1, 1, tn2), lambda j, i: (i, 0, j)),
                  pl.BlockSpec((1, 1, tn2), lambda j, i: (i, 0, j))],
        out_specs=pl.BlockSpec((L, tn2), lambda j, i: (0, j)),
        out_shape=lcol,
    )(xv, wk3, ik3)

    return jnp.transpose(out_cols.reshape(L, B, D), (1, 0, 2))
